# Initial kernel scaffold; baseline (speedup 1.0000x reference)
import jax
import jax.numpy as jnp
from jax.experimental import pallas as pl


def kernel(feat, edge_index, edge_affine, W, b):
    raise NotImplementedError("write your pallas kernel here")


# speedup vs baseline: 4.3984x; 4.3984x over previous
import functools

import jax
import jax.numpy as jnp
from jax import lax
from jax.experimental import pallas as pl
from jax.experimental.pallas import tpu as pltpu
from jax.experimental.pallas import tpu_sc as plsc

N_NODES = 10000
N_EDGES = 320000
D = 128
LANES = 16

NC = 2
NS = 16
NW = NC * NS

E_PER_W = N_EDGES // NW
CHUNK = 80
NCHUNK = E_PER_W // CHUNK
R_SLICE = 624
R_TAIL_BASE = NS * R_SLICE
R_TAIL = N_NODES - R_TAIL_BASE


def _sc_aggregate(src, dst, aff, feat, zeros):
    mesh = plsc.VectorSubcoreMesh(core_axis_name="c", subcore_axis_name="s")

    @functools.partial(
        pl.kernel,
        mesh=mesh,
        out_type=jax.ShapeDtypeStruct((NC * N_NODES, D), jnp.float32),
        scratch_types=[
            pltpu.VMEM((CHUNK,), jnp.int32),
            pltpu.VMEM((CHUNK,), jnp.int32),
            pltpu.VMEM((CHUNK,), jnp.float32),
            pltpu.VMEM((CHUNK, D), jnp.float32),
            pltpu.VMEM_SHARED((N_NODES, D), jnp.float32),
            pltpu.SemaphoreType.DMA,
        ],
    )
    def sc_kernel(src_hbm, dst_hbm, aff_hbm, feat_hbm, zeros_hbm, out_hbm,
                  src_v, dst_v, aff_v, rows_v, agg_sh, sem):
        c = lax.axis_index("c")
        s = lax.axis_index("s")
        wid = s * NC + c

        pltpu.sync_copy(
            zeros_hbm.at[pl.ds(s * R_SLICE, R_SLICE)],
            agg_sh.at[pl.ds(s * R_SLICE, R_SLICE)])

        @pl.when(s == 0)
        def _():
            pltpu.sync_copy(
                zeros_hbm.at[pl.ds(R_TAIL_BASE, R_TAIL)],
                agg_sh.at[pl.ds(R_TAIL_BASE, R_TAIL)])

        plsc.subcore_barrier()

        ebase = wid * E_PER_W

        def chunk_body(ci, carry):
            base = ebase + ci * CHUNK
            pltpu.sync_copy(src_hbm.at[pl.ds(base, CHUNK)], src_v)
            pltpu.sync_copy(dst_hbm.at[pl.ds(base, CHUNK)], dst_v)
            pltpu.sync_copy(aff_hbm.at[pl.ds(base, CHUNK)], aff_v)
            pltpu.async_copy(feat_hbm.at[src_v], rows_v, sem).wait()

            def grp_body(g, gcarry):
                a = aff_v[pl.ds(g * LANES, LANES)]
                for l in range(LANES):
                    e = g * LANES + l
                    av = a[l]
                    for j in range(D // LANES):
                        sl = pl.ds(j * LANES, LANES)
                        rows_v[e, sl] = rows_v[e, sl] * av
                return gcarry

            lax.fori_loop(0, CHUNK // LANES, grp_body, 0)
            pltpu.sync_copy(rows_v, agg_sh.at[dst_v], add=True)
            return carry

        lax.fori_loop(0, NCHUNK, chunk_body, 0)
        plsc.subcore_barrier()

        rbase = s * R_SLICE
        pltpu.sync_copy(
            agg_sh.at[pl.ds(rbase, R_SLICE)],
            out_hbm.at[pl.ds(c * N_NODES + rbase, R_SLICE)])

        @pl.when(s == 0)
        def _():
            pltpu.sync_copy(
                agg_sh.at[pl.ds(R_TAIL_BASE, R_TAIL)],
                out_hbm.at[pl.ds(c * N_NODES + R_TAIL_BASE, R_TAIL)])

    return sc_kernel(src, dst, aff, feat, zeros)


_TC_BLK = 1000


def _tc_body(feat_ref, agg0_ref, agg1_ref, w1t_ref, w2t_ref, b_ref, out_ref):
    acc = jnp.dot(feat_ref[...], w1t_ref[...],
                  preferred_element_type=jnp.float32)
    agg = agg0_ref[...] + agg1_ref[...]
    acc = acc + jnp.dot(agg, w2t_ref[...],
                        preferred_element_type=jnp.float32)
    out_ref[...] = acc + b_ref[...]


def _tc_linear(feat, agg0, agg1, w1t, w2t, b2d):
    grid = (N_NODES // _TC_BLK,)
    return pl.pallas_call(
        _tc_body,
        grid=grid,
        in_specs=[
            pl.BlockSpec((_TC_BLK, D), lambda i: (i, 0)),
            pl.BlockSpec((_TC_BLK, D), lambda i: (i, 0)),
            pl.BlockSpec((_TC_BLK, D), lambda i: (i, 0)),
            pl.BlockSpec((D, D), lambda i: (0, 0)),
            pl.BlockSpec((D, D), lambda i: (0, 0)),
            pl.BlockSpec((1, D), lambda i: (0, 0)),
        ],
        out_specs=pl.BlockSpec((_TC_BLK, D), lambda i: (i, 0)),
        out_shape=jax.ShapeDtypeStruct((N_NODES, D), jnp.float32),
    )(feat, agg0, agg1, w1t, w2t, b2d)


def kernel(feat, edge_index, edge_affine, W, b):
    src = edge_index[0]
    dst = edge_index[1]
    zeros = jnp.zeros((N_NODES, D), jnp.float32)
    partials = _sc_aggregate(src, dst, edge_affine, feat, zeros)
    agg0 = partials[:N_NODES]
    agg1 = partials[N_NODES:]
    w1t = W[:, :D].T
    w2t = W[:, D:].T
    return _tc_linear(feat, agg0, agg1, w1t, w2t, b.reshape(1, D))


# speedup vs baseline: 10.3256x; 2.3476x over previous
import functools

import jax
import jax.numpy as jnp
from jax import lax
from jax.experimental import pallas as pl
from jax.experimental.pallas import tpu as pltpu
from jax.experimental.pallas import tpu_sc as plsc

N_NODES = 10000
N_EDGES = 320000
D = 128
LANES = 16

NC = 2
NS = 16
NW = NC * NS

E_PER_W = N_EDGES // NW
CHUNK = 80
NCHUNK = E_PER_W // CHUNK
NPAIR = (NCHUNK - 1) // 2
R_SLICE = 624
R_TAIL_BASE = NS * R_SLICE
R_TAIL = N_NODES - R_TAIL_BASE


def _sc_aggregate(src, dst, aff, feat, zeros):
    mesh = plsc.VectorSubcoreMesh(core_axis_name="c", subcore_axis_name="s")

    @functools.partial(
        pl.kernel,
        mesh=mesh,
        out_type=jax.ShapeDtypeStruct((NC * N_NODES, D), jnp.float32),
        scratch_types=[
            pltpu.VMEM((CHUNK,), jnp.int32),
            pltpu.VMEM((CHUNK,), jnp.int32),
            pltpu.VMEM((CHUNK,), jnp.float32),
            pltpu.VMEM((CHUNK,), jnp.float32),
            pltpu.VMEM((NCHUNK, CHUNK), jnp.int32),
            pltpu.VMEM((CHUNK, D), jnp.float32),
            pltpu.VMEM((CHUNK, D), jnp.float32),
            pltpu.VMEM_SHARED((N_NODES, D), jnp.float32),
            pltpu.SemaphoreType.DMA,
            pltpu.SemaphoreType.DMA,
            pltpu.SemaphoreType.DMA,
            pltpu.SemaphoreType.DMA,
            pltpu.SemaphoreType.DMA,
        ],
    )
    def sc_kernel(src_hbm, dst_hbm, aff_hbm, feat_hbm, zeros_hbm, out_hbm,
                  src_a, src_b, aff_a, aff_b, dst_v, rows0, rows1, agg_sh,
                  sem0, sem1, sem_s, sem_aa, sem_ab):
        c = lax.axis_index("c")
        s = lax.axis_index("s")
        wid = s * NC + c

        pltpu.sync_copy(
            zeros_hbm.at[pl.ds(s * R_SLICE, R_SLICE)],
            agg_sh.at[pl.ds(s * R_SLICE, R_SLICE)])

        @pl.when(s == 0)
        def _():
            pltpu.sync_copy(
                zeros_hbm.at[pl.ds(R_TAIL_BASE, R_TAIL)],
                agg_sh.at[pl.ds(R_TAIL_BASE, R_TAIL)])

        pltpu.sync_copy(dst_hbm.at[wid], dst_v)
        plsc.subcore_barrier()

        ebase = wid * E_PER_W

        def load_src(ci, buf):
            return pltpu.async_copy(
                src_hbm.at[pl.ds(ebase + ci * CHUNK, CHUNK)], buf, sem_s)

        def load_aff(ci, buf, sem_b):
            return pltpu.async_copy(
                aff_hbm.at[pl.ds(ebase + ci * CHUNK, CHUNK)], buf, sem_b)

        def wait_aff(buf, sem_b):
            pltpu.make_async_copy(
                aff_hbm.at[pl.ds(0, CHUNK)], buf, sem_b).wait()

        def gather(src_buf, rows_b, sem_b):
            return pltpu.async_copy(feat_hbm.at[src_buf], rows_b, sem_b)

        def wait_gather(rows_b, sem_b):
            pltpu.make_async_copy(
                feat_hbm.at[pl.ds(0, CHUNK)], rows_b, sem_b).wait()

        def scale_scatter(ci, rows_b, aff_buf):
            def grp_body(g, gcarry):
                a = aff_buf[pl.ds(g * LANES, LANES)]
                for l in range(LANES):
                    e = g * LANES + l
                    av = a[l]
                    for j in range(D // LANES):
                        sl = pl.ds(j * LANES, LANES)
                        rows_b[e, sl] = rows_b[e, sl] * av
                return gcarry

            lax.fori_loop(0, CHUNK // LANES, grp_body, 0)
            pltpu.sync_copy(rows_b, agg_sh.at[dst_v.at[ci]], add=True)

        load_aff(0, aff_a, sem_aa)
        load_aff(1, aff_b, sem_ab)
        load_src(0, src_a).wait()
        gather(src_a, rows0, sem0)
        load_src(1, src_b).wait()

        def pair_body(p, carry):
            ci = p * 2
            gather(src_b, rows1, sem1)
            d_a = load_src(ci + 2, src_a)
            wait_gather(rows0, sem0)
            wait_aff(aff_a, sem_aa)
            scale_scatter(ci, rows0, aff_a)
            load_aff(ci + 2, aff_a, sem_aa)
            d_a.wait()
            gather(src_a, rows0, sem0)
            d_b = load_src(ci + 3, src_b)
            wait_gather(rows1, sem1)
            wait_aff(aff_b, sem_ab)
            scale_scatter(ci + 1, rows1, aff_b)
            load_aff(ci + 3, aff_b, sem_ab)
            d_b.wait()
            return carry

        lax.fori_loop(0, NPAIR, pair_body, 0)
        wait_gather(rows0, sem0)
        wait_aff(aff_a, sem_aa)
        scale_scatter(NCHUNK - 1, rows0, aff_a)
        wait_aff(aff_b, sem_ab)

        plsc.subcore_barrier()
        rbase = s * R_SLICE
        pltpu.sync_copy(
            agg_sh.at[pl.ds(rbase, R_SLICE)],
            out_hbm.at[pl.ds(c * N_NODES + rbase, R_SLICE)])

        @pl.when(s == 0)
        def _():
            pltpu.sync_copy(
                agg_sh.at[pl.ds(R_TAIL_BASE, R_TAIL)],
                out_hbm.at[pl.ds(c * N_NODES + R_TAIL_BASE, R_TAIL)])

    return sc_kernel(src, dst, aff, feat, zeros)


_TC_BLK = 1000


def _tc_body(feat_ref, agg0_ref, agg1_ref, w1t_ref, w2t_ref, b_ref, out_ref):
    acc = jnp.dot(feat_ref[...], w1t_ref[...],
                  preferred_element_type=jnp.float32)
    agg = agg0_ref[...] + agg1_ref[...]
    acc = acc + jnp.dot(agg, w2t_ref[...],
                        preferred_element_type=jnp.float32)
    out_ref[...] = acc + b_ref[...]


def _tc_linear(feat, agg0, agg1, w1t, w2t, b2d):
    grid = (N_NODES // _TC_BLK,)
    return pl.pallas_call(
        _tc_body,
        grid=grid,
        in_specs=[
            pl.BlockSpec((_TC_BLK, D), lambda i: (i, 0)),
            pl.BlockSpec((_TC_BLK, D), lambda i: (i, 0)),
            pl.BlockSpec((_TC_BLK, D), lambda i: (i, 0)),
            pl.BlockSpec((D, D), lambda i: (0, 0)),
            pl.BlockSpec((D, D), lambda i: (0, 0)),
            pl.BlockSpec((1, D), lambda i: (0, 0)),
        ],
        out_specs=pl.BlockSpec((_TC_BLK, D), lambda i: (i, 0)),
        out_shape=jax.ShapeDtypeStruct((N_NODES, D), jnp.float32),
    )(feat, agg0, agg1, w1t, w2t, b2d)


def kernel(feat, edge_index, edge_affine, W, b):
    src = jnp.pad(edge_index[0], (0, CHUNK))
    dst = edge_index[1].reshape(NW, NCHUNK, CHUNK)
    aff = jnp.pad(edge_affine, (0, 2 * CHUNK))
    zeros = jnp.zeros((N_NODES, D), jnp.float32)
    partials = _sc_aggregate(src, dst, aff, feat, zeros)
    agg0 = partials[:N_NODES]
    agg1 = partials[N_NODES:]
    w1t = W[:, :D].T
    w2t = W[:, D:].T
    return _tc_linear(feat, agg0, agg1, w1t, w2t, b.reshape(1, D))


# speedup vs baseline: 12.1748x; 1.1791x over previous
import functools

import jax
import jax.numpy as jnp
from jax import lax
from jax.experimental import pallas as pl
from jax.experimental.pallas import tpu as pltpu
from jax.experimental.pallas import tpu_sc as plsc

N_NODES = 10000
N_EDGES = 320000
D = 128
LANES = 16

NC = 2
NS = 16
NW = NC * NS

E_PER_W = N_EDGES // NW
CHUNK = 80
NCHUNK = E_PER_W // CHUNK
NBODY = (NCHUNK - 5) // 3
R_SLICE = 624
R_TAIL_BASE = NS * R_SLICE
R_TAIL = N_NODES - R_TAIL_BASE


def _sc_aggregate(edge_flat, dst, aff, feat):
    mesh = plsc.VectorSubcoreMesh(core_axis_name="c", subcore_axis_name="s")

    @functools.partial(
        pl.kernel,
        mesh=mesh,
        out_type=jax.ShapeDtypeStruct((NC * N_NODES, D), jnp.float32),
        scratch_types=(
            [pltpu.VMEM((CHUNK,), jnp.int32) for _ in range(3)]
            + [pltpu.VMEM((CHUNK,), jnp.float32) for _ in range(3)]
            + [pltpu.VMEM((CHUNK, D), jnp.float32) for _ in range(3)]
            + [pltpu.VMEM((NCHUNK, CHUNK), jnp.int32)]
            + [pltpu.VMEM_SHARED((N_NODES, D), jnp.float32)]
            + [pltpu.SemaphoreType.DMA for _ in range(12)]
        ),
    )
    def sc_kernel(edge_hbm, dst_hbm, aff_hbm, feat_hbm, out_hbm,
                  s0, s1, s2, a0, a1, a2, r0, r1, r2, dst_v, agg_sh,
                  *sems):
        srcb = [s0, s1, s2]
        affb = [a0, a1, a2]
        rows = [r0, r1, r2]
        sem_s = sems[0:3]
        sem_a = sems[3:6]
        sem_g = sems[6:9]
        sem_c = sems[9:12]

        c = lax.axis_index("c")
        s = lax.axis_index("s")
        wid = s * NC + c

        def zfill(e, zcarry):
            for j in range(D // LANES):
                r0[e, pl.ds(j * LANES, LANES)] = jnp.zeros(
                    (LANES,), jnp.float32)
            return zcarry

        lax.fori_loop(0, CHUNK, zfill, 0)
        zbase = s * R_SLICE
        for k in range(7):
            pltpu.sync_copy(r0, agg_sh.at[pl.ds(zbase + k * CHUNK, CHUNK)])
        pltpu.sync_copy(r0.at[pl.ds(0, R_SLICE - 7 * CHUNK)],
                        agg_sh.at[pl.ds(zbase + 7 * CHUNK,
                                        R_SLICE - 7 * CHUNK)])

        @pl.when(s == 0)
        def _():
            pltpu.sync_copy(r0.at[pl.ds(0, R_TAIL)],
                            agg_sh.at[pl.ds(R_TAIL_BASE, R_TAIL)])

        pltpu.sync_copy(dst_hbm.at[wid], dst_v)
        plsc.subcore_barrier()

        ebase = wid * E_PER_W

        def load_src(ci, k):
            return pltpu.async_copy(
                edge_hbm.at[pl.ds(ebase + ci * CHUNK, CHUNK)], srcb[k],
                sem_s[k])

        def wait_src(k):
            pltpu.make_async_copy(edge_hbm.at[pl.ds(0, CHUNK)], srcb[k],
                                  sem_s[k]).wait()

        def load_aff(ci, k):
            return pltpu.async_copy(
                aff_hbm.at[pl.ds(ebase + ci * CHUNK, CHUNK)], affb[k],
                sem_a[k])

        def wait_aff(k):
            pltpu.make_async_copy(aff_hbm.at[pl.ds(0, CHUNK)], affb[k],
                                  sem_a[k]).wait()

        def gather(k_src, k_rows):
            return pltpu.async_copy(feat_hbm.at[srcb[k_src]], rows[k_rows],
                                    sem_g[k_rows])

        def wait_gather(k):
            pltpu.make_async_copy(feat_hbm.at[pl.ds(0, CHUNK)], rows[k],
                                  sem_g[k]).wait()

        def wait_scatter(k):
            pltpu.make_async_copy(rows[k], agg_sh.at[pl.ds(0, CHUNK)],
                                  sem_c[k]).wait()

        def scale(ci, k):
            def grp_body(g, gcarry):
                a = affb[k][pl.ds(g * LANES, LANES)]
                for l in range(LANES):
                    e = g * LANES + l
                    av = a[l]
                    for j in range(D // LANES):
                        sl = pl.ds(j * LANES, LANES)
                        rows[k][e, sl] = rows[k][e, sl] * av
                return gcarry

            lax.fori_loop(0, CHUNK // LANES, grp_body, 0)

        def step(ci, k, wait_sc=True, gath=True, pre=True):
            k2 = (k + 2) % 3
            wait_gather(k)
            wait_aff(k)
            scale(ci, k)
            if pre:
                load_aff(ci + 3, k)
            if wait_sc:
                wait_scatter(k2)
            pltpu.async_copy(rows[k], agg_sh.at[dst_v.at[ci]], sem_c[k],
                             add=True)
            if gath:
                wait_src(k2)
                gather(k2, k2)
                if pre:
                    load_src(ci + 3, k)

        for k in range(3):
            load_src(k, k)
            load_aff(k, k)
        wait_src(0)
        gather(0, 0)
        wait_src(1)
        gather(1, 1)

        step(0, 0, wait_sc=False)
        step(1, 1)

        def body(q, carry):
            ci = 3 * q + 2
            step(ci, 2)
            step(ci + 1, 0)
            step(ci + 2, 1)
            return carry

        lax.fori_loop(0, NBODY, body, 0)
        step(122, 2, pre=False)
        step(123, 0, gath=False, pre=False)
        step(124, 1, gath=False, pre=False)
        wait_scatter(1)

        plsc.subcore_barrier()
        rbase = s * R_SLICE
        pltpu.sync_copy(
            agg_sh.at[pl.ds(rbase, R_SLICE)],
            out_hbm.at[pl.ds(c * N_NODES + rbase, R_SLICE)])

        @pl.when(s == 0)
        def _():
            pltpu.sync_copy(
                agg_sh.at[pl.ds(R_TAIL_BASE, R_TAIL)],
                out_hbm.at[pl.ds(c * N_NODES + R_TAIL_BASE, R_TAIL)])

    return sc_kernel(edge_flat, dst, aff, feat)


_TC_BLK = 2000


def _tc_body(feat_ref, agg0_ref, agg1_ref, w_ref, b_ref, out_ref):
    w1 = w_ref[:, :D]
    w2 = w_ref[:, D:]
    dims = (((1,), (1,)), ((), ()))
    acc = lax.dot_general(feat_ref[...], w1, dims,
                          preferred_element_type=jnp.float32)
    agg = agg0_ref[...] + agg1_ref[...]
    acc = acc + lax.dot_general(agg, w2, dims,
                                preferred_element_type=jnp.float32)
    out_ref[...] = acc + b_ref[...]


def _tc_linear(feat, partials, W, b2d):
    nblk = N_NODES // _TC_BLK
    grid = (nblk,)
    return pl.pallas_call(
        _tc_body,
        grid=grid,
        in_specs=[
            pl.BlockSpec((_TC_BLK, D), lambda i: (i, 0)),
            pl.BlockSpec((_TC_BLK, D), lambda i: (i, 0)),
            pl.BlockSpec((_TC_BLK, D), lambda i: (i + nblk, 0)),
            pl.BlockSpec((D, 2 * D), lambda i: (0, 0)),
            pl.BlockSpec((1, D), lambda i: (0, 0)),
        ],
        out_specs=pl.BlockSpec((_TC_BLK, D), lambda i: (i, 0)),
        out_shape=jax.ShapeDtypeStruct((N_NODES, D), jnp.float32),
    )(feat, partials, partials, W, b2d)


def kernel(feat, edge_index, edge_affine, W, b):
    edge_flat = edge_index.reshape(2 * N_EDGES)
    dst = edge_index[1].reshape(NW, NCHUNK, CHUNK)
    partials = _sc_aggregate(edge_flat, dst, edge_affine, feat)
    return _tc_linear(feat, partials, W, b.reshape(1, D))
